# Initial kernel scaffold; baseline (speedup 1.0000x reference)
import functools

import jax
import jax.numpy as jnp
from jax import lax
from jax.experimental import pallas as pl
from jax.experimental.pallas import tpu as pltpu
from jax.experimental.pallas import tpu_sc as plsc

_NUM_OUT = 10000
_NUM_IN = 320000
_D = 128
_NNZ = 640000

_NC = 2
_NS = 16
_NW = _NC * _NS
_CH = 128
_CHUNKS = -(-_NNZ // (_NW * _CH))
_EDGES_PER_W = _CHUNKS * _CH
_NPAD = _EDGES_PER_W * _NW
_ROWS_PER_TILE = _NUM_OUT // _NS
_RCH = 125


def _sc_body(feat_hbm, rows_hbm, cols_hbm, vals_hbm, out_hbm,
             cols_v, rows_v, vals_v, gbuf, acc_sh, gsem):
    c = lax.axis_index("c")
    s = lax.axis_index("s")
    wid = c * _NS + s
    edge_base = wid * _EDGES_PER_W

    def _zero_row(e, _):
        zero = jnp.zeros((16,), jnp.float32)
        for j in range(_D // 16):
            gbuf[e, pl.ds(j * 16, 16)] = zero
        return 0
    lax.fori_loop(0, _RCH, _zero_row, 0)
    row0 = s * _ROWS_PER_TILE
    for i in range(_ROWS_PER_TILE // _RCH):
        pltpu.sync_copy(gbuf.at[pl.ds(0, _RCH)],
                        acc_sh.at[pl.ds(row0 + i * _RCH, _RCH)])
    plsc.subcore_barrier()

    def _chunk(ci, _):
        base = edge_base + ci * _CH
        pltpu.sync_copy(cols_hbm.at[pl.ds(base, _CH)], cols_v)
        pltpu.sync_copy(rows_hbm.at[pl.ds(base, _CH)], rows_v)
        pltpu.sync_copy(vals_hbm.at[pl.ds(base, _CH)], vals_v)
        pltpu.async_copy(feat_hbm.at[cols_v], gbuf, gsem).wait()

        def _scale(e, _):
            vv = plsc.load_gather(vals_v, [jnp.full((16,), e, jnp.int32)])
            for j in range(_D // 16):
                sl = pl.ds(j * 16, 16)
                gbuf[e, sl] = gbuf[e, sl] * vv
            return 0
        lax.fori_loop(0, _CH, _scale, 0)

        pltpu.sync_copy(gbuf, acc_sh.at[rows_v], add=True)
        return 0
    lax.fori_loop(0, _CHUNKS, _chunk, 0)

    plsc.subcore_barrier()
    for i in range(_ROWS_PER_TILE // _RCH):
        r0 = row0 + i * _RCH
        pltpu.sync_copy(acc_sh.at[pl.ds(r0, _RCH)], gbuf.at[pl.ds(0, _RCH)])
        pltpu.sync_copy(gbuf.at[pl.ds(0, _RCH)], out_hbm.at[c, pl.ds(r0, _RCH)])


_sc_spmm = functools.partial(
    pl.kernel,
    out_type=jax.ShapeDtypeStruct((_NC, _NUM_OUT, _D), jnp.float32),
    mesh=plsc.VectorSubcoreMesh(core_axis_name="c", subcore_axis_name="s"),
    scratch_types=[
        pltpu.VMEM((_CH,), jnp.int32),
        pltpu.VMEM((_CH,), jnp.int32),
        pltpu.VMEM((_CH,), jnp.float32),
        pltpu.VMEM((_CH, _D), jnp.float32),
        pltpu.VMEM_SHARED((_NUM_OUT, _D), jnp.float32),
        pltpu.SemaphoreType.DMA,
    ],
)(_sc_body)


def _sum2_body(p_ref, o_ref):
    o_ref[...] = p_ref[0] + p_ref[1]


def _sum_partials(partials):
    blk = 1000
    return pl.pallas_call(
        _sum2_body,
        grid=(_NUM_OUT // blk,),
        in_specs=[pl.BlockSpec((_NC, blk, _D), lambda i: (0, i, 0))],
        out_specs=pl.BlockSpec((blk, _D), lambda i: (i, 0)),
        out_shape=jax.ShapeDtypeStruct((_NUM_OUT, _D), jnp.float32),
    )(partials)


@jax.jit
def kernel(simplex_features, boundary_indices, boundary_values):
    rows = boundary_indices[0].astype(jnp.int32)
    cols = boundary_indices[1].astype(jnp.int32)
    vals = boundary_values.astype(jnp.float32)
    pad = _NPAD - _NNZ
    rows = jnp.concatenate([rows, jnp.zeros((pad,), jnp.int32)])
    cols = jnp.concatenate([cols, jnp.zeros((pad,), jnp.int32)])
    vals = jnp.concatenate([vals, jnp.zeros((pad,), jnp.float32)])
    partials = _sc_spmm(simplex_features, rows, cols, vals)
    return _sum_partials(partials)


# speedup vs baseline: 2.7941x; 2.7941x over previous
import functools

import jax
import jax.numpy as jnp
from jax import lax
from jax.experimental import pallas as pl
from jax.experimental.pallas import tpu as pltpu
from jax.experimental.pallas import tpu_sc as plsc

_NUM_OUT = 10000
_NUM_IN = 320000
_D = 128
_NNZ = 640000

_NC = 2
_NS = 16
_NW = _NC * _NS
_CH = 128
_CHUNKS = -(-_NNZ // (_NW * _CH))
_EDGES_PER_W = _CHUNKS * _CH
_NPAD = _EDGES_PER_W * _NW
_ROW_STRIDE = 624
_RCH = 128
_RCOPIES = 5


def _sc_body(feat_hbm, rows_hbm, cols_hbm, vals_hbm, out_hbm,
             cols_v, rows_v, vals_v, gbuf, acc_sh, gsem):
    c = lax.axis_index("c")
    s = lax.axis_index("s")
    wid = c * _NS + s
    edge_base = wid * _EDGES_PER_W

    def _zero_row(e, _):
        zero = jnp.zeros((16,), jnp.float32)
        for j in range(_D // 16):
            gbuf[e, pl.ds(j * 16, 16)] = zero
        return 0
    lax.fori_loop(0, _RCH, _zero_row, 0)
    row0 = s * _ROW_STRIDE
    for i in range(_RCOPIES):
        pltpu.sync_copy(gbuf, acc_sh.at[pl.ds(row0 + i * _RCH, _RCH)])
    plsc.subcore_barrier()

    def _chunk(ci, _):
        base = edge_base + ci * _CH
        pltpu.sync_copy(cols_hbm.at[pl.ds(base, _CH)], cols_v)
        pltpu.sync_copy(rows_hbm.at[pl.ds(base, _CH)], rows_v)
        pltpu.sync_copy(vals_hbm.at[pl.ds(base, _CH)], vals_v.at[pl.ds(0, _CH)])
        pltpu.async_copy(feat_hbm.at[cols_v], gbuf, gsem).wait()

        def _scale(e, _):
            vv = jnp.full((16,), vals_v[pl.ds(e, 16)][0], jnp.float32)
            for j in range(_D // 16):
                sl = pl.ds(j * 16, 16)
                gbuf[e, sl] = gbuf[e, sl] * vv
            return 0
        lax.fori_loop(0, _CH, _scale, 0)

        pltpu.sync_copy(gbuf, acc_sh.at[rows_v], add=True)
        return 0
    lax.fori_loop(0, _CHUNKS, _chunk, 0)

    plsc.subcore_barrier()
    for i in range(_RCOPIES):
        r0 = row0 + i * _RCH
        pltpu.sync_copy(acc_sh.at[pl.ds(r0, _RCH)], gbuf)
        pltpu.sync_copy(gbuf, out_hbm.at[c, pl.ds(r0, _RCH)])


_sc_spmm = functools.partial(
    pl.kernel,
    out_type=jax.ShapeDtypeStruct((_NC, _NUM_OUT, _D), jnp.float32),
    mesh=plsc.VectorSubcoreMesh(core_axis_name="c", subcore_axis_name="s"),
    scratch_types=[
        pltpu.VMEM((_CH,), jnp.int32),
        pltpu.VMEM((_CH,), jnp.int32),
        pltpu.VMEM((_CH + 16,), jnp.float32),
        pltpu.VMEM((_CH, _D), jnp.float32),
        pltpu.VMEM_SHARED((_NUM_OUT, _D), jnp.float32),
        pltpu.SemaphoreType.DMA,
    ],
)(_sc_body)


def _sum2_body(p_ref, o_ref):
    o_ref[...] = p_ref[0] + p_ref[1]


def _sum_partials(partials):
    blk = 1000
    return pl.pallas_call(
        _sum2_body,
        grid=(_NUM_OUT // blk,),
        in_specs=[pl.BlockSpec((_NC, blk, _D), lambda i: (0, i, 0))],
        out_specs=pl.BlockSpec((blk, _D), lambda i: (i, 0)),
        out_shape=jax.ShapeDtypeStruct((_NUM_OUT, _D), jnp.float32),
    )(partials)


@jax.jit
def kernel(simplex_features, boundary_indices, boundary_values):
    rows = boundary_indices[0].astype(jnp.int32)
    cols = boundary_indices[1].astype(jnp.int32)
    vals = boundary_values.astype(jnp.float32)
    pad = _NPAD - _NNZ
    rows = jnp.concatenate([rows, jnp.zeros((pad,), jnp.int32)])
    cols = jnp.concatenate([cols, jnp.zeros((pad,), jnp.int32)])
    vals = jnp.concatenate([vals, jnp.zeros((pad,), jnp.float32)])
    partials = _sc_spmm(simplex_features, rows, cols, vals)
    return _sum_partials(partials)
